# baseline (speedup 1.0000x reference)
import functools

import jax
import jax.numpy as jnp
from jax import lax
from jax.experimental import pallas as pl
from jax.experimental.pallas import tpu as pltpu
from jax.experimental.pallas import tpu_sc as plsc

_B, _NQ, _D = 16, 1024, 256
_H, _W, _NH, _NP = 32, 32, 8, 4
_HD = _D // _NH
_NPC = _NP * 4
_NC, _NS = 2, 16
_NWORK = _NC * _NS
_PAIRS_PER_W = (_B * _NH) // _NWORK


def _prep_body(q_ref, v_ref, wval_ref, bval_ref, woff_ref, boff_ref,
               wattn_ref, battn_ref, vh_ref, idx_ref, wgt_ref):
    qb = q_ref[0]
    vb = v_ref[0]

    for h in range(_NH):
        wv_h = wval_ref[h * _HD:(h + 1) * _HD, :]
        vh = lax.dot_general(vb, wv_h, (((1,), (1,)), ((), ())),
                             preferred_element_type=jnp.float32)
        vh = vh + bval_ref[h]
        for c in range(4):
            vh_ref[h * 256:(h + 1) * 256, c * _HD:(c + 1) * _HD] = (
                vh[c * 256:(c + 1) * 256, :])

    offr = (lax.dot_general(woff_ref[...], qb, (((1,), (1,)), ((), ())),
                            preferred_element_type=jnp.float32)
            + boff_ref[...])
    off = jnp.tanh(offr)
    awr = (lax.dot_general(wattn_ref[...], qb, (((1,), (1,)), ((), ())),
                           preferred_element_type=jnp.float32)
           + battn_ref[...])

    aws = [awr[p * _NH:(p + 1) * _NH] for p in range(_NP)]
    m = jnp.maximum(jnp.maximum(aws[0], aws[1]), jnp.maximum(aws[2], aws[3]))
    es = [jnp.exp(a - m) for a in aws]
    rs = 1.0 / (es[0] + es[1] + es[2] + es[3])

    qi = lax.broadcasted_iota(jnp.int32, (_NH, _NQ), 1)
    gx = (qi % _W).astype(jnp.float32) * (2.0 / (_W - 1)) - 1.0
    gy = (qi // _W).astype(jnp.float32) * (2.0 / (_H - 1)) - 1.0

    for p in range(_NP):
        offx = off[p * _NH:(p + 1) * _NH]
        offy = off[32 + p * _NH:32 + (p + 1) * _NH]
        awn = es[p] * rs
        locx = jnp.clip(gx + 0.5 * offx, -1.0, 1.0)
        locy = jnp.clip(gy + 0.5 * offy, -1.0, 1.0)
        x = (locx + 1.0) * (_W / 2.0) - 0.5
        y = (locy + 1.0) * (_H / 2.0) - 0.5
        x0f = jnp.floor(x)
        y0f = jnp.floor(y)
        wx1 = x - x0f
        wy1 = y - y0f
        ix0 = x0f.astype(jnp.int32)
        iy0 = y0f.astype(jnp.int32)
        for c, (cy, cx) in enumerate(((0, 0), (0, 1), (1, 0), (1, 1))):
            ix = ix0 + cx
            iy = iy0 + cy
            wx = wx1 if cx else 1.0 - wx1
            wy = wy1 if cy else 1.0 - wy1
            valid = ((ix >= 0) & (ix <= _W - 1) & (iy >= 0) & (iy <= _H - 1))
            idxc = jnp.clip(iy, 0, _H - 1) * _W + jnp.clip(ix, 0, _W - 1)
            wc = wx * wy * awn * valid.astype(jnp.float32)
            pc = c * _NP + p
            idx_ref[:, pc, :] = idxc
            wgt_ref[:, pc, :] = wc


def _out_body(sh_ref, wout_ref, bout_ref, o_ref):
    acc = bout_ref[...]
    out = None
    for h in range(_NH):
        sh = jnp.concatenate(
            [sh_ref[h * 256:(h + 1) * 256, c * _HD:(c + 1) * _HD]
             for c in range(4)], axis=0)
        part = lax.dot_general(sh, wout_ref[h],
                               (((1,), (1,)), ((), ())),
                               preferred_element_type=jnp.float32)
        out = part if out is None else out + part
    o_ref[0] = out + acc


def _sc_body(vh_hbm, idx_hbm, wgt_hbm, out_hbm, table, idxs, wgts, outv):
    wid = lax.axis_index("c") * _NS + lax.axis_index("s")

    def pair_body(k, carry):
        e = wid * _PAIRS_PER_W + k
        r0 = pl.multiple_of(e * (_NQ * _HD), 8)
        pltpu.sync_copy(vh_hbm.at[pl.ds(r0, _NQ * _HD)], table)
        pltpu.sync_copy(idx_hbm.at[e], idxs)
        pltpu.sync_copy(wgt_hbm.at[e], wgts)

        def q_body(qb, qcarry):
            q0 = pl.multiple_of(qb * 16, 16)
            rows_v = [idxs[pc, pl.ds(q0, 16)] for pc in range(_NPC)]
            w_v = [wgts[pc, pl.ds(q0, 16)] for pc in range(_NPC)]
            qc = qb // 16
            ob = pl.multiple_of((q0 - qc * 256) * 128 + qc * _HD, 16)
            for u in range(16):
                acc0 = jnp.zeros((16,), jnp.float32)
                acc1 = jnp.zeros((16,), jnp.float32)
                for pc in range(_NPC):
                    rr = rows_v[pc][u]
                    w = w_v[pc][u]
                    rc = rr // 256
                    base = pl.multiple_of((rr - rc * 256) * 128 + rc * _HD, 16)
                    g0 = table[pl.ds(base, 16)]
                    g1 = table[pl.ds(base + 16, 16)]
                    acc0 = acc0 + w * g0
                    acc1 = acc1 + w * g1
                outv[pl.ds(ob + u * 128, 16)] = acc0
                outv[pl.ds(ob + u * 128 + 16, 16)] = acc1
            return qcarry

        lax.fori_loop(0, _NQ // 16, q_body, 0)
        pltpu.sync_copy(outv, out_hbm.at[pl.ds(r0, _NQ * _HD)])
        return carry

    lax.fori_loop(0, _PAIRS_PER_W, pair_body, 0)


def _sc_gather(vh, idx, wgt):
    mesh = plsc.VectorSubcoreMesh(core_axis_name="c", subcore_axis_name="s",
                                  num_cores=_NC, num_subcores=_NS)
    return pl.kernel(
        _sc_body,
        out_type=jax.ShapeDtypeStruct((_B * _NH * _NQ * _HD,), jnp.float32),
        mesh=mesh,
        scratch_types=[
            pltpu.VMEM((_NQ * _HD,), jnp.float32),
            pltpu.VMEM((_NPC, _NQ), jnp.int32),
            pltpu.VMEM((_NPC, _NQ), jnp.float32),
            pltpu.VMEM((_NQ * _HD,), jnp.float32),
        ],
        compiler_params=pltpu.CompilerParams(needs_layout_passes=False),
    )(vh.reshape(-1), idx, wgt)


def _prep_call(query, value, W_val, b_val_r, W_off_r, b_off_r, W_attn_r,
               b_attn_r, *, interpret=False):
    full = lambda shape: pl.BlockSpec(shape, lambda b: (0,) * len(shape))
    return pl.pallas_call(
        _prep_body,
        grid=(_B,),
        in_specs=[
            pl.BlockSpec((1, _NQ, _D), lambda b: (b, 0, 0)),
            pl.BlockSpec((1, _NQ, _D), lambda b: (b, 0, 0)),
            full((_D, _D)),
            full((_NH, 1, _HD)),
            full((2 * _NP * _NH, _D)),
            full((2 * _NP * _NH, 1)),
            full((_NP * _NH, _D)),
            full((_NP * _NH, 1)),
        ],
        out_specs=[
            pl.BlockSpec((_NH * 256, 128), lambda b: (b, 0)),
            pl.BlockSpec((_NH, _NPC, _NQ), lambda b: (b, 0, 0)),
            pl.BlockSpec((_NH, _NPC, _NQ), lambda b: (b, 0, 0)),
        ],
        out_shape=[
            jax.ShapeDtypeStruct((_B * _NH * 256, 128), jnp.float32),
            jax.ShapeDtypeStruct((_B * _NH, _NPC, _NQ), jnp.int32),
            jax.ShapeDtypeStruct((_B * _NH, _NPC, _NQ), jnp.float32),
        ],
        interpret=interpret,
    )(query, value, W_val, b_val_r, W_off_r, b_off_r, W_attn_r, b_attn_r)


def _out_call(sh, W_out_r, b_out_r, *, interpret=False):
    return pl.pallas_call(
        _out_body,
        grid=(_B,),
        in_specs=[
            pl.BlockSpec((_NH * 256, 128), lambda b: (b, 0)),
            pl.BlockSpec((_NH, _D, _HD), lambda b: (0, 0, 0)),
            pl.BlockSpec((1, _D), lambda b: (0, 0)),
        ],
        out_specs=pl.BlockSpec((1, _NQ, _D), lambda b: (b, 0, 0)),
        out_shape=jax.ShapeDtypeStruct((_B, _NQ, _D), jnp.float32),
        interpret=interpret,
    )(sh, W_out_r, b_out_r)


def kernel(query, value, W_off, b_off, W_attn, b_attn, W_val, b_val, W_out,
           b_out, spatial_shape, *, interpret=False):
    W_off_r = W_off.reshape(_NH, _NP, 2, _D).transpose(2, 1, 0, 3).reshape(2 * _NP * _NH, _D)
    b_off_r = b_off.reshape(_NH, _NP, 2).transpose(2, 1, 0).reshape(2 * _NP * _NH, 1)
    W_attn_r = W_attn.reshape(_NH, _NP, _D).transpose(1, 0, 2).reshape(_NP * _NH, _D)
    b_attn_r = b_attn.reshape(_NH, _NP).transpose(1, 0).reshape(_NP * _NH, 1)
    b_val_r = b_val.reshape(_NH, 1, _HD)
    W_out_r = W_out.reshape(_D, _NH, _HD).transpose(1, 0, 2)
    b_out_r = b_out.reshape(1, _D)

    vh, idx, wgt = _prep_call(query, value, W_val, b_val_r, W_off_r, b_off_r,
                              W_attn_r, b_attn_r, interpret=interpret)

    sh = _sc_gather(vh, idx, wgt).reshape(_B * _NH * 256, 128)

    return _out_call(sh, W_out_r, b_out_r, interpret=interpret)


# speedup vs baseline: 3.2709x; 3.2709x over previous
import functools

import jax
import jax.numpy as jnp
from jax import lax
from jax.experimental import pallas as pl
from jax.experimental.pallas import tpu as pltpu
from jax.experimental.pallas import tpu_sc as plsc

_B, _NQ, _D = 16, 1024, 256
_H, _W, _NH, _NP = 32, 32, 8, 4
_HD = _D // _NH
_NPC = _NP * 4
_NC, _NS = 2, 16
_NWORK = _NC * _NS
_PAIRS_PER_W = (_B * _NH) // _NWORK


def _prep_body(q_ref, v_ref, wval_ref, bval_ref, woff_ref, boff_ref,
               wattn_ref, battn_ref, vh_ref, idx_ref, wgt_ref):
    qb = q_ref[0]
    vb = v_ref[0]

    for h in range(_NH):
        wv_h = wval_ref[h * _HD:(h + 1) * _HD, :]
        vh = lax.dot_general(vb, wv_h, (((1,), (1,)), ((), ())),
                             preferred_element_type=jnp.float32)
        vh = vh + bval_ref[h]
        for c in range(4):
            vh_ref[h * 256:(h + 1) * 256, c * _HD:(c + 1) * _HD] = (
                vh[c * 256:(c + 1) * 256, :])

    offr = (lax.dot_general(woff_ref[...], qb, (((1,), (1,)), ((), ())),
                            preferred_element_type=jnp.float32)
            + boff_ref[...])
    off = jnp.tanh(offr)
    awr = (lax.dot_general(wattn_ref[...], qb, (((1,), (1,)), ((), ())),
                           preferred_element_type=jnp.float32)
           + battn_ref[...])

    aws = [awr[p * _NH:(p + 1) * _NH] for p in range(_NP)]
    m = jnp.maximum(jnp.maximum(aws[0], aws[1]), jnp.maximum(aws[2], aws[3]))
    es = [jnp.exp(a - m) for a in aws]
    rs = 1.0 / (es[0] + es[1] + es[2] + es[3])

    qi = lax.broadcasted_iota(jnp.int32, (_NH, _NQ), 1)
    gx = (qi % _W).astype(jnp.float32) * (2.0 / (_W - 1)) - 1.0
    gy = (qi // _W).astype(jnp.float32) * (2.0 / (_H - 1)) - 1.0

    for p in range(_NP):
        offx = off[p * _NH:(p + 1) * _NH]
        offy = off[32 + p * _NH:32 + (p + 1) * _NH]
        awn = es[p] * rs
        locx = jnp.clip(gx + 0.5 * offx, -1.0, 1.0)
        locy = jnp.clip(gy + 0.5 * offy, -1.0, 1.0)
        x = (locx + 1.0) * (_W / 2.0) - 0.5
        y = (locy + 1.0) * (_H / 2.0) - 0.5
        x0f = jnp.floor(x)
        y0f = jnp.floor(y)
        wx1 = x - x0f
        wy1 = y - y0f
        ix0 = x0f.astype(jnp.int32)
        iy0 = y0f.astype(jnp.int32)
        for c, (cy, cx) in enumerate(((0, 0), (0, 1), (1, 0), (1, 1))):
            ix = ix0 + cx
            iy = iy0 + cy
            wx = wx1 if cx else 1.0 - wx1
            wy = wy1 if cy else 1.0 - wy1
            valid = ((ix >= 0) & (ix <= _W - 1) & (iy >= 0) & (iy <= _H - 1))
            idxc = jnp.clip(iy, 0, _H - 1) * _W + jnp.clip(ix, 0, _W - 1)
            wc = wx * wy * awn * valid.astype(jnp.float32)
            pc = c * _NP + p
            idx_ref[:, pc, :] = idxc
            wgt_ref[:, pc, :] = wc


def _out_body(sh_ref, wout_ref, bout_ref, o_ref):
    acc = bout_ref[...]
    out = None
    for h in range(_NH):
        sh = jnp.concatenate(
            [sh_ref[h * 256:(h + 1) * 256, c * _HD:(c + 1) * _HD]
             for c in range(4)], axis=0)
        part = lax.dot_general(sh, wout_ref[h],
                               (((1,), (1,)), ((), ())),
                               preferred_element_type=jnp.float32)
        out = part if out is None else out + part
    o_ref[0] = out + acc


def _sc_body(vh_hbm, idx_hbm, wgt_hbm, out_hbm, table, idxs, wgts, outv):
    wid = lax.axis_index("c") * _NS + lax.axis_index("s")

    def pair_body(k, carry):
        e = wid * _PAIRS_PER_W + k
        r0 = pl.multiple_of(e * (_NQ * _HD), 8)
        pltpu.sync_copy(vh_hbm.at[pl.ds(r0, _NQ * _HD)], table)
        pltpu.sync_copy(idx_hbm.at[e], idxs)
        pltpu.sync_copy(wgt_hbm.at[e], wgts)

        def q_body(qb, qcarry):
            q0 = pl.multiple_of(qb * 16, 16)
            rows_v = [idxs[pc, pl.ds(q0, 16)] for pc in range(_NPC)]
            w_v = [wgts[pc, pl.ds(q0, 16)] for pc in range(_NPC)]
            qc = qb >> 4
            ob = pl.multiple_of(((q0 - (qc << 8)) << 7) + (qc << 5), 16)
            for u in range(16):
                acc0 = jnp.zeros((16,), jnp.float32)
                acc1 = jnp.zeros((16,), jnp.float32)
                for pc in range(_NPC):
                    rr = rows_v[pc][u]
                    w = w_v[pc][u]
                    base = pl.multiple_of(
                        ((rr & 255) << 7) + ((rr >> 8) << 5), 16)
                    g0 = table[pl.ds(base, 16)]
                    g1 = table[pl.ds(base + 16, 16)]
                    acc0 = acc0 + w * g0
                    acc1 = acc1 + w * g1
                outv[pl.ds(ob + u * 128, 16)] = acc0
                outv[pl.ds(ob + u * 128 + 16, 16)] = acc1
            return qcarry

        lax.fori_loop(0, _NQ // 16, q_body, 0)
        pltpu.sync_copy(outv, out_hbm.at[pl.ds(r0, _NQ * _HD)])
        return carry

    lax.fori_loop(0, _PAIRS_PER_W, pair_body, 0)


def _sc_gather(vh, idx, wgt):
    mesh = plsc.VectorSubcoreMesh(core_axis_name="c", subcore_axis_name="s",
                                  num_cores=_NC, num_subcores=_NS)
    return pl.kernel(
        _sc_body,
        out_type=jax.ShapeDtypeStruct((_B * _NH * _NQ * _HD,), jnp.float32),
        mesh=mesh,
        scratch_types=[
            pltpu.VMEM((_NQ * _HD,), jnp.float32),
            pltpu.VMEM((_NPC, _NQ), jnp.int32),
            pltpu.VMEM((_NPC, _NQ), jnp.float32),
            pltpu.VMEM((_NQ * _HD,), jnp.float32),
        ],
        compiler_params=pltpu.CompilerParams(needs_layout_passes=False),
    )(vh.reshape(-1), idx, wgt)


def _prep_call(query, value, W_val, b_val_r, W_off_r, b_off_r, W_attn_r,
               b_attn_r, *, interpret=False):
    full = lambda shape: pl.BlockSpec(shape, lambda b: (0,) * len(shape))
    return pl.pallas_call(
        _prep_body,
        grid=(_B,),
        in_specs=[
            pl.BlockSpec((1, _NQ, _D), lambda b: (b, 0, 0)),
            pl.BlockSpec((1, _NQ, _D), lambda b: (b, 0, 0)),
            full((_D, _D)),
            full((_NH, 1, _HD)),
            full((2 * _NP * _NH, _D)),
            full((2 * _NP * _NH, 1)),
            full((_NP * _NH, _D)),
            full((_NP * _NH, 1)),
        ],
        out_specs=[
            pl.BlockSpec((_NH * 256, 128), lambda b: (b, 0)),
            pl.BlockSpec((_NH, _NPC, _NQ), lambda b: (b, 0, 0)),
            pl.BlockSpec((_NH, _NPC, _NQ), lambda b: (b, 0, 0)),
        ],
        out_shape=[
            jax.ShapeDtypeStruct((_B * _NH * 256, 128), jnp.float32),
            jax.ShapeDtypeStruct((_B * _NH, _NPC, _NQ), jnp.int32),
            jax.ShapeDtypeStruct((_B * _NH, _NPC, _NQ), jnp.float32),
        ],
        interpret=interpret,
    )(query, value, W_val, b_val_r, W_off_r, b_off_r, W_attn_r, b_attn_r)


def _out_call(sh, W_out_r, b_out_r, *, interpret=False):
    return pl.pallas_call(
        _out_body,
        grid=(_B,),
        in_specs=[
            pl.BlockSpec((_NH * 256, 128), lambda b: (b, 0)),
            pl.BlockSpec((_NH, _D, _HD), lambda b: (0, 0, 0)),
            pl.BlockSpec((1, _D), lambda b: (0, 0)),
        ],
        out_specs=pl.BlockSpec((1, _NQ, _D), lambda b: (b, 0, 0)),
        out_shape=jax.ShapeDtypeStruct((_B, _NQ, _D), jnp.float32),
        interpret=interpret,
    )(sh, W_out_r, b_out_r)


def kernel(query, value, W_off, b_off, W_attn, b_attn, W_val, b_val, W_out,
           b_out, spatial_shape, *, interpret=False):
    W_off_r = W_off.reshape(_NH, _NP, 2, _D).transpose(2, 1, 0, 3).reshape(2 * _NP * _NH, _D)
    b_off_r = b_off.reshape(_NH, _NP, 2).transpose(2, 1, 0).reshape(2 * _NP * _NH, 1)
    W_attn_r = W_attn.reshape(_NH, _NP, _D).transpose(1, 0, 2).reshape(_NP * _NH, _D)
    b_attn_r = b_attn.reshape(_NH, _NP).transpose(1, 0).reshape(_NP * _NH, 1)
    b_val_r = b_val.reshape(_NH, 1, _HD)
    W_out_r = W_out.reshape(_D, _NH, _HD).transpose(1, 0, 2)
    b_out_r = b_out.reshape(1, _D)

    vh, idx, wgt = _prep_call(query, value, W_val, b_val_r, W_off_r, b_off_r,
                              W_attn_r, b_attn_r, interpret=interpret)

    sh = _sc_gather(vh, idx, wgt).reshape(_B * _NH * 256, 128)

    return _out_call(sh, W_out_r, b_out_r, interpret=interpret)


# speedup vs baseline: 3.5762x; 1.0933x over previous
import functools

import jax
import jax.numpy as jnp
from jax import lax
from jax.experimental import pallas as pl
from jax.experimental.pallas import tpu as pltpu
from jax.experimental.pallas import tpu_sc as plsc

_B, _NQ, _D = 16, 1024, 256
_H, _W, _NH, _NP = 32, 32, 8, 4
_HD = _D // _NH
_NPC = _NP * 4
_NC, _NS = 2, 16
_NWORK = _NC * _NS
_PAIRS_PER_W = (_B * _NH) // _NWORK


def _prep_body(q_ref, v_ref, wval_ref, bval_ref, woff_ref, boff_ref,
               wattn_ref, battn_ref, vh_ref, idx_ref, wgt_ref):
    qb = q_ref[0]
    vb = v_ref[0]

    for h in range(_NH):
        wv_h = wval_ref[h * _HD:(h + 1) * _HD, :]
        vh = lax.dot_general(vb, wv_h, (((1,), (1,)), ((), ())),
                             preferred_element_type=jnp.float32)
        vh = vh + bval_ref[h]
        for c in range(4):
            vh_ref[h * 256:(h + 1) * 256, c * _HD:(c + 1) * _HD] = (
                vh[c * 256:(c + 1) * 256, :])

    offr = (lax.dot_general(woff_ref[...], qb, (((1,), (1,)), ((), ())),
                            preferred_element_type=jnp.float32)
            + boff_ref[...])
    off = jnp.tanh(offr)
    awr = (lax.dot_general(wattn_ref[...], qb, (((1,), (1,)), ((), ())),
                           preferred_element_type=jnp.float32)
           + battn_ref[...])

    aws = [awr[p * _NH:(p + 1) * _NH] for p in range(_NP)]
    m = jnp.maximum(jnp.maximum(aws[0], aws[1]), jnp.maximum(aws[2], aws[3]))
    es = [jnp.exp(a - m) for a in aws]
    rs = 1.0 / (es[0] + es[1] + es[2] + es[3])

    qi = lax.broadcasted_iota(jnp.int32, (_NH, _NQ), 1)
    gx = (qi % _W).astype(jnp.float32) * (2.0 / (_W - 1)) - 1.0
    gy = (qi // _W).astype(jnp.float32) * (2.0 / (_H - 1)) - 1.0

    for p in range(_NP):
        offx = off[p * _NH:(p + 1) * _NH]
        offy = off[32 + p * _NH:32 + (p + 1) * _NH]
        awn = es[p] * rs
        locx = jnp.clip(gx + 0.5 * offx, -1.0, 1.0)
        locy = jnp.clip(gy + 0.5 * offy, -1.0, 1.0)
        x = (locx + 1.0) * (_W / 2.0) - 0.5
        y = (locy + 1.0) * (_H / 2.0) - 0.5
        x0f = jnp.floor(x)
        y0f = jnp.floor(y)
        wx1 = x - x0f
        wy1 = y - y0f
        ix0 = x0f.astype(jnp.int32)
        iy0 = y0f.astype(jnp.int32)
        for c, (cy, cx) in enumerate(((0, 0), (0, 1), (1, 0), (1, 1))):
            ix = ix0 + cx
            iy = iy0 + cy
            wx = wx1 if cx else 1.0 - wx1
            wy = wy1 if cy else 1.0 - wy1
            valid = ((ix >= 0) & (ix <= _W - 1) & (iy >= 0) & (iy <= _H - 1))
            idxc = jnp.clip(iy, 0, _H - 1) * _W + jnp.clip(ix, 0, _W - 1)
            wc = wx * wy * awn * valid.astype(jnp.float32)
            pc = c * _NP + p
            idx_ref[:, pc, :] = (((idxc & 255) << 7) + ((idxc >> 8) << 5))
            wgt_ref[:, pc, :] = wc


def _out_body(sh_ref, wout_ref, bout_ref, o_ref):
    acc = bout_ref[...]
    out = None
    for h in range(_NH):
        sh = jnp.concatenate(
            [sh_ref[h * 256:(h + 1) * 256, c * _HD:(c + 1) * _HD]
             for c in range(4)], axis=0)
        part = lax.dot_general(sh, wout_ref[h],
                               (((1,), (1,)), ((), ())),
                               preferred_element_type=jnp.float32)
        out = part if out is None else out + part
    o_ref[0] = out + acc


def _sc_body(vh_hbm, idx_hbm, wgt_hbm, out_hbm, table, idxs, wgts, outv):
    wid = lax.axis_index("c") * _NS + lax.axis_index("s")

    def pair_body(k, carry):
        e = wid * _PAIRS_PER_W + k
        r0 = pl.multiple_of(e * (_NQ * _HD), 8)
        pltpu.sync_copy(vh_hbm.at[pl.ds(r0, _NQ * _HD)], table)
        pltpu.sync_copy(idx_hbm.at[e], idxs)
        pltpu.sync_copy(wgt_hbm.at[e], wgts)

        def q_body(qb, qcarry):
            q0 = pl.multiple_of(qb * 16, 16)
            rows_v = [idxs[pc, pl.ds(q0, 16)] for pc in range(_NPC)]
            w_v = [wgts[pc, pl.ds(q0, 16)] for pc in range(_NPC)]
            qc = qb >> 4
            ob = pl.multiple_of(((q0 - (qc << 8)) << 7) + (qc << 5), 16)
            for u in range(16):
                acc0 = jnp.zeros((16,), jnp.float32)
                acc1 = jnp.zeros((16,), jnp.float32)
                for pc in range(_NPC):
                    base = pl.multiple_of(rows_v[pc][u], 16)
                    w = w_v[pc][u]
                    g0 = table[pl.ds(base, 16)]
                    g1 = table[pl.ds(base + 16, 16)]
                    acc0 = acc0 + w * g0
                    acc1 = acc1 + w * g1
                outv[pl.ds(ob + u * 128, 16)] = acc0
                outv[pl.ds(ob + u * 128 + 16, 16)] = acc1
            return qcarry

        lax.fori_loop(0, _NQ // 16, q_body, 0)
        pltpu.sync_copy(outv, out_hbm.at[pl.ds(r0, _NQ * _HD)])
        return carry

    lax.fori_loop(0, _PAIRS_PER_W, pair_body, 0)


def _sc_gather(vh, idx, wgt):
    mesh = plsc.VectorSubcoreMesh(core_axis_name="c", subcore_axis_name="s",
                                  num_cores=_NC, num_subcores=_NS)
    return pl.kernel(
        _sc_body,
        out_type=jax.ShapeDtypeStruct((_B * _NH * _NQ * _HD,), jnp.float32),
        mesh=mesh,
        scratch_types=[
            pltpu.VMEM((_NQ * _HD,), jnp.float32),
            pltpu.VMEM((_NPC, _NQ), jnp.int32),
            pltpu.VMEM((_NPC, _NQ), jnp.float32),
            pltpu.VMEM((_NQ * _HD,), jnp.float32),
        ],
        compiler_params=pltpu.CompilerParams(needs_layout_passes=False),
    )(vh.reshape(-1), idx, wgt)


def _prep_call(query, value, W_val, b_val_r, W_off_r, b_off_r, W_attn_r,
               b_attn_r, *, interpret=False):
    full = lambda shape: pl.BlockSpec(shape, lambda b: (0,) * len(shape))
    return pl.pallas_call(
        _prep_body,
        grid=(_B,),
        in_specs=[
            pl.BlockSpec((1, _NQ, _D), lambda b: (b, 0, 0)),
            pl.BlockSpec((1, _NQ, _D), lambda b: (b, 0, 0)),
            full((_D, _D)),
            full((_NH, 1, _HD)),
            full((2 * _NP * _NH, _D)),
            full((2 * _NP * _NH, 1)),
            full((_NP * _NH, _D)),
            full((_NP * _NH, 1)),
        ],
        out_specs=[
            pl.BlockSpec((_NH * 256, 128), lambda b: (b, 0)),
            pl.BlockSpec((_NH, _NPC, _NQ), lambda b: (b, 0, 0)),
            pl.BlockSpec((_NH, _NPC, _NQ), lambda b: (b, 0, 0)),
        ],
        out_shape=[
            jax.ShapeDtypeStruct((_B * _NH * 256, 128), jnp.float32),
            jax.ShapeDtypeStruct((_B * _NH, _NPC, _NQ), jnp.int32),
            jax.ShapeDtypeStruct((_B * _NH, _NPC, _NQ), jnp.float32),
        ],
        interpret=interpret,
    )(query, value, W_val, b_val_r, W_off_r, b_off_r, W_attn_r, b_attn_r)


def _out_call(sh, W_out_r, b_out_r, *, interpret=False):
    return pl.pallas_call(
        _out_body,
        grid=(_B,),
        in_specs=[
            pl.BlockSpec((_NH * 256, 128), lambda b: (b, 0)),
            pl.BlockSpec((_NH, _D, _HD), lambda b: (0, 0, 0)),
            pl.BlockSpec((1, _D), lambda b: (0, 0)),
        ],
        out_specs=pl.BlockSpec((1, _NQ, _D), lambda b: (b, 0, 0)),
        out_shape=jax.ShapeDtypeStruct((_B, _NQ, _D), jnp.float32),
        interpret=interpret,
    )(sh, W_out_r, b_out_r)


def kernel(query, value, W_off, b_off, W_attn, b_attn, W_val, b_val, W_out,
           b_out, spatial_shape, *, interpret=False):
    W_off_r = W_off.reshape(_NH, _NP, 2, _D).transpose(2, 1, 0, 3).reshape(2 * _NP * _NH, _D)
    b_off_r = b_off.reshape(_NH, _NP, 2).transpose(2, 1, 0).reshape(2 * _NP * _NH, 1)
    W_attn_r = W_attn.reshape(_NH, _NP, _D).transpose(1, 0, 2).reshape(_NP * _NH, _D)
    b_attn_r = b_attn.reshape(_NH, _NP).transpose(1, 0).reshape(_NP * _NH, 1)
    b_val_r = b_val.reshape(_NH, 1, _HD)
    W_out_r = W_out.reshape(_D, _NH, _HD).transpose(1, 0, 2)
    b_out_r = b_out.reshape(1, _D)

    vh, idx, wgt = _prep_call(query, value, W_val, b_val_r, W_off_r, b_off_r,
                              W_attn_r, b_attn_r, interpret=interpret)

    sh = _sc_gather(vh, idx, wgt).reshape(_B * _NH * 256, 128)

    return _out_call(sh, W_out_r, b_out_r, interpret=interpret)


# speedup vs baseline: 3.7228x; 1.0410x over previous
import functools

import jax
import jax.numpy as jnp
from jax import lax
from jax.experimental import pallas as pl
from jax.experimental.pallas import tpu as pltpu
from jax.experimental.pallas import tpu_sc as plsc

_B, _NQ, _D = 16, 1024, 256
_H, _W, _NH, _NP = 32, 32, 8, 4
_HD = _D // _NH
_NPC = _NP * 4
_NC, _NS = 2, 16
_NWORK = _NC * _NS
_PAIRS_PER_W = (_B * _NH) // _NWORK


def _prep_body(q_ref, v_ref, wval_ref, bval_ref, woff_ref, boff_ref,
               wattn_ref, battn_ref, vh_ref, idx_ref, wgt_ref):
    qb = q_ref[0]
    vb = v_ref[0]

    for h in range(_NH):
        wv_h = wval_ref[h * _HD:(h + 1) * _HD, :]
        vh = lax.dot_general(vb, wv_h, (((1,), (1,)), ((), ())),
                             preferred_element_type=jnp.float32)
        vh = vh + bval_ref[h]
        for c in range(4):
            vh_ref[h * 256:(h + 1) * 256, c * _HD:(c + 1) * _HD] = (
                vh[c * 256:(c + 1) * 256, :])

    offr = (lax.dot_general(woff_ref[...], qb, (((1,), (1,)), ((), ())),
                            preferred_element_type=jnp.float32)
            + boff_ref[...])
    off = jnp.tanh(offr)
    awr = (lax.dot_general(wattn_ref[...], qb, (((1,), (1,)), ((), ())),
                           preferred_element_type=jnp.float32)
           + battn_ref[...])

    aws = [awr[p * _NH:(p + 1) * _NH] for p in range(_NP)]
    m = jnp.maximum(jnp.maximum(aws[0], aws[1]), jnp.maximum(aws[2], aws[3]))
    es = [jnp.exp(a - m) for a in aws]
    rs = 1.0 / (es[0] + es[1] + es[2] + es[3])

    qi = lax.broadcasted_iota(jnp.int32, (_NH, _NQ), 1)
    gx = (qi % _W).astype(jnp.float32) * (2.0 / (_W - 1)) - 1.0
    gy = (qi // _W).astype(jnp.float32) * (2.0 / (_H - 1)) - 1.0

    for p in range(_NP):
        offx = off[p * _NH:(p + 1) * _NH]
        offy = off[32 + p * _NH:32 + (p + 1) * _NH]
        awn = es[p] * rs
        locx = jnp.clip(gx + 0.5 * offx, -1.0, 1.0)
        locy = jnp.clip(gy + 0.5 * offy, -1.0, 1.0)
        x = (locx + 1.0) * (_W / 2.0) - 0.5
        y = (locy + 1.0) * (_H / 2.0) - 0.5
        x0f = jnp.floor(x)
        y0f = jnp.floor(y)
        wx1 = x - x0f
        wy1 = y - y0f
        ix0 = x0f.astype(jnp.int32)
        iy0 = y0f.astype(jnp.int32)
        for c, (cy, cx) in enumerate(((0, 0), (0, 1), (1, 0), (1, 1))):
            ix = ix0 + cx
            iy = iy0 + cy
            wx = wx1 if cx else 1.0 - wx1
            wy = wy1 if cy else 1.0 - wy1
            valid = ((ix >= 0) & (ix <= _W - 1) & (iy >= 0) & (iy <= _H - 1))
            idxc = jnp.clip(iy, 0, _H - 1) * _W + jnp.clip(ix, 0, _W - 1)
            wc = wx * wy * awn * valid.astype(jnp.float32)
            pc = c * _NP + p
            idx_ref[:, pc, :] = (((idxc & 255) << 7) + ((idxc >> 8) << 5))
            wgt_ref[:, pc, :] = wc


def _out_body(sh_ref, wout_ref, bout_ref, o_ref):
    acc = bout_ref[...]
    out = None
    for h in range(_NH):
        sh = jnp.concatenate(
            [sh_ref[h * 256:(h + 1) * 256, c * _HD:(c + 1) * _HD]
             for c in range(4)], axis=0)
        part = lax.dot_general(sh, wout_ref[h],
                               (((1,), (1,)), ((), ())),
                               preferred_element_type=jnp.float32)
        out = part if out is None else out + part
    o_ref[0] = out + acc


def _sc_body(ppw, vh_hbm, idx_hbm, wgt_hbm, out_hbm, table, idxs, wgts, outv):
    wid = lax.axis_index("c") * _NS + lax.axis_index("s")

    def pair_body(k, carry):
        e = wid * ppw + k
        r0 = pl.multiple_of(e * (_NQ * _HD), 8)
        pltpu.sync_copy(vh_hbm.at[pl.ds(r0, _NQ * _HD)], table)
        pltpu.sync_copy(idx_hbm.at[e], idxs)
        pltpu.sync_copy(wgt_hbm.at[e], wgts)

        def q_body(qb, qcarry):
            q0 = pl.multiple_of(qb * 16, 16)
            rows_v = [idxs[pc, pl.ds(q0, 16)] for pc in range(_NPC)]
            w_v = [wgts[pc, pl.ds(q0, 16)] for pc in range(_NPC)]
            qc = qb >> 4
            ob = pl.multiple_of(((q0 - (qc << 8)) << 7) + (qc << 5), 16)
            for u in range(16):
                acc0 = jnp.zeros((16,), jnp.float32)
                acc1 = jnp.zeros((16,), jnp.float32)
                for pc in range(_NPC):
                    base = pl.multiple_of(rows_v[pc][u], 16)
                    w = w_v[pc][u]
                    g0 = table[pl.ds(base, 16)]
                    g1 = table[pl.ds(base + 16, 16)]
                    acc0 = acc0 + w * g0
                    acc1 = acc1 + w * g1
                outv[pl.ds(ob + u * 128, 16)] = acc0
                outv[pl.ds(ob + u * 128 + 16, 16)] = acc1
            return qcarry

        lax.fori_loop(0, _NQ // 16, q_body, 0)
        pltpu.sync_copy(outv, out_hbm.at[pl.ds(r0, _NQ * _HD)])
        return carry

    lax.fori_loop(0, ppw, pair_body, 0)


def _sc_gather(vh, idx, wgt, nb):
    mesh = plsc.VectorSubcoreMesh(core_axis_name="c", subcore_axis_name="s",
                                  num_cores=_NC, num_subcores=_NS)
    ppw = (nb * _NH) // _NWORK
    return pl.kernel(
        functools.partial(_sc_body, ppw),
        out_type=jax.ShapeDtypeStruct((nb * _NH * _NQ * _HD,), jnp.float32),
        mesh=mesh,
        scratch_types=[
            pltpu.VMEM((_NQ * _HD,), jnp.float32),
            pltpu.VMEM((_NPC, _NQ), jnp.int32),
            pltpu.VMEM((_NPC, _NQ), jnp.float32),
            pltpu.VMEM((_NQ * _HD,), jnp.float32),
        ],
        compiler_params=pltpu.CompilerParams(needs_layout_passes=False),
    )(vh.reshape(-1), idx, wgt)


def _prep_call(query, value, W_val, b_val_r, W_off_r, b_off_r, W_attn_r,
               b_attn_r, b0, nb, *, interpret=False):
    full = lambda shape: pl.BlockSpec(shape, lambda b: (0,) * len(shape))
    return pl.pallas_call(
        _prep_body,
        grid=(nb,),
        in_specs=[
            pl.BlockSpec((1, _NQ, _D), lambda b: (b + b0, 0, 0)),
            pl.BlockSpec((1, _NQ, _D), lambda b: (b + b0, 0, 0)),
            full((_D, _D)),
            full((_NH, 1, _HD)),
            full((2 * _NP * _NH, _D)),
            full((2 * _NP * _NH, 1)),
            full((_NP * _NH, _D)),
            full((_NP * _NH, 1)),
        ],
        out_specs=[
            pl.BlockSpec((_NH * 256, 128), lambda b: (b, 0)),
            pl.BlockSpec((_NH, _NPC, _NQ), lambda b: (b, 0, 0)),
            pl.BlockSpec((_NH, _NPC, _NQ), lambda b: (b, 0, 0)),
        ],
        out_shape=[
            jax.ShapeDtypeStruct((nb * _NH * 256, 128), jnp.float32),
            jax.ShapeDtypeStruct((nb * _NH, _NPC, _NQ), jnp.int32),
            jax.ShapeDtypeStruct((nb * _NH, _NPC, _NQ), jnp.float32),
        ],
        interpret=interpret,
    )(query, value, W_val, b_val_r, W_off_r, b_off_r, W_attn_r, b_attn_r)


def _out_call(sh, W_out_r, b_out_r, nb, *, interpret=False):
    return pl.pallas_call(
        _out_body,
        grid=(nb,),
        in_specs=[
            pl.BlockSpec((_NH * 256, 128), lambda b: (b, 0)),
            pl.BlockSpec((_NH, _D, _HD), lambda b: (0, 0, 0)),
            pl.BlockSpec((1, _D), lambda b: (0, 0)),
        ],
        out_specs=pl.BlockSpec((1, _NQ, _D), lambda b: (b, 0, 0)),
        out_shape=jax.ShapeDtypeStruct((nb, _NQ, _D), jnp.float32),
        interpret=interpret,
    )(sh, W_out_r, b_out_r)


def kernel(query, value, W_off, b_off, W_attn, b_attn, W_val, b_val, W_out,
           b_out, spatial_shape, *, interpret=False):
    W_off_r = W_off.reshape(_NH, _NP, 2, _D).transpose(2, 1, 0, 3).reshape(2 * _NP * _NH, _D)
    b_off_r = b_off.reshape(_NH, _NP, 2).transpose(2, 1, 0).reshape(2 * _NP * _NH, 1)
    W_attn_r = W_attn.reshape(_NH, _NP, _D).transpose(1, 0, 2).reshape(_NP * _NH, _D)
    b_attn_r = b_attn.reshape(_NH, _NP).transpose(1, 0).reshape(_NP * _NH, 1)
    b_val_r = b_val.reshape(_NH, 1, _HD)
    W_out_r = W_out.reshape(_D, _NH, _HD).transpose(1, 0, 2)
    b_out_r = b_out.reshape(1, _D)

    ngroups = 2
    nb = _B // ngroups
    outs = []
    for g in range(ngroups):
        vh, idx, wgt = _prep_call(query, value, W_val, b_val_r, W_off_r,
                                  b_off_r, W_attn_r, b_attn_r, g * nb, nb,
                                  interpret=interpret)
        sh = _sc_gather(vh, idx, wgt, nb).reshape(nb * _NH * 256, 128)
        outs.append(_out_call(sh, W_out_r, b_out_r, nb, interpret=interpret))
    return jnp.concatenate(outs, axis=0)


# speedup vs baseline: 3.7624x; 1.0106x over previous
import functools

import jax
import jax.numpy as jnp
from jax import lax
from jax.experimental import pallas as pl
from jax.experimental.pallas import tpu as pltpu
from jax.experimental.pallas import tpu_sc as plsc

_B, _NQ, _D = 16, 1024, 256
_H, _W, _NH, _NP = 32, 32, 8, 4
_HD = _D // _NH
_NPC = _NP * 4
_NC, _NS = 2, 16
_NWORK = _NC * _NS
_PAIRS_PER_W = (_B * _NH) // _NWORK


def _prep_body(q_ref, v_ref, wval_ref, bval_ref, woff_ref, boff_ref,
               wattn_ref, battn_ref, vh_ref, idx_ref, wgt_ref):
    qb = q_ref[0]
    vb = v_ref[0]

    for h in range(_NH):
        wv_h = wval_ref[h * _HD:(h + 1) * _HD, :]
        vh = lax.dot_general(vb, wv_h, (((1,), (1,)), ((), ())),
                             preferred_element_type=jnp.float32)
        vh = vh + bval_ref[h]
        for c in range(4):
            vh_ref[h * 256:(h + 1) * 256, c * _HD:(c + 1) * _HD] = (
                vh[c * 256:(c + 1) * 256, :])

    offr = (lax.dot_general(woff_ref[...], qb, (((1,), (1,)), ((), ())),
                            preferred_element_type=jnp.float32)
            + boff_ref[...])
    off = jnp.tanh(offr)
    awr = (lax.dot_general(wattn_ref[...], qb, (((1,), (1,)), ((), ())),
                           preferred_element_type=jnp.float32)
           + battn_ref[...])

    aws = [awr[p * _NH:(p + 1) * _NH] for p in range(_NP)]
    m = jnp.maximum(jnp.maximum(aws[0], aws[1]), jnp.maximum(aws[2], aws[3]))
    es = [jnp.exp(a - m) for a in aws]
    rs = 1.0 / (es[0] + es[1] + es[2] + es[3])

    qi = lax.broadcasted_iota(jnp.int32, (_NH, _NQ), 1)
    gx = (qi % _W).astype(jnp.float32) * (2.0 / (_W - 1)) - 1.0
    gy = (qi // _W).astype(jnp.float32) * (2.0 / (_H - 1)) - 1.0

    for p in range(_NP):
        offx = off[p * _NH:(p + 1) * _NH]
        offy = off[32 + p * _NH:32 + (p + 1) * _NH]
        awn = es[p] * rs
        locx = jnp.clip(gx + 0.5 * offx, -1.0, 1.0)
        locy = jnp.clip(gy + 0.5 * offy, -1.0, 1.0)
        x = (locx + 1.0) * (_W / 2.0) - 0.5
        y = (locy + 1.0) * (_H / 2.0) - 0.5
        x0f = jnp.floor(x)
        y0f = jnp.floor(y)
        wx1 = x - x0f
        wy1 = y - y0f
        ix0 = x0f.astype(jnp.int32)
        iy0 = y0f.astype(jnp.int32)
        for c, (cy, cx) in enumerate(((0, 0), (0, 1), (1, 0), (1, 1))):
            ix = ix0 + cx
            iy = iy0 + cy
            wx = wx1 if cx else 1.0 - wx1
            wy = wy1 if cy else 1.0 - wy1
            valid = ((ix >= 0) & (ix <= _W - 1) & (iy >= 0) & (iy <= _H - 1))
            idxc = jnp.clip(iy, 0, _H - 1) * _W + jnp.clip(ix, 0, _W - 1)
            wc = wx * wy * awn * valid.astype(jnp.float32)
            pc = c * _NP + p
            idx_ref[:, pc, :] = (((idxc & 255) << 7) + ((idxc >> 8) << 5))
            wgt_ref[:, pc, :] = wc


def _out_body(sh_ref, wout_ref, bout_ref, o_ref):
    acc = bout_ref[...]
    out = None
    for h in range(_NH):
        sh = jnp.concatenate(
            [sh_ref[h * 256:(h + 1) * 256, c * _HD:(c + 1) * _HD]
             for c in range(4)], axis=0)
        part = lax.dot_general(sh, wout_ref[h],
                               (((1,), (1,)), ((), ())),
                               preferred_element_type=jnp.float32)
        out = part if out is None else out + part
    o_ref[0] = out + acc


def _sc_body(ppw, vh_hbm, idx_hbm, wgt_hbm, out_hbm, table, idxs, wgts, outv):
    wid = lax.axis_index("c") * _NS + lax.axis_index("s")

    def pair_body(k, carry):
        e = wid * ppw + k
        r0 = pl.multiple_of(e * (_NQ * _HD), 8)
        pltpu.sync_copy(vh_hbm.at[pl.ds(r0, _NQ * _HD)], table)
        pltpu.sync_copy(idx_hbm.at[e], idxs)
        pltpu.sync_copy(wgt_hbm.at[e], wgts)

        def q_body(qb, qcarry):
            q0 = pl.multiple_of(qb * 16, 16)
            rows_v = [idxs[pc, pl.ds(q0, 16)] for pc in range(_NPC)]
            w_v = [wgts[pc, pl.ds(q0, 16)] for pc in range(_NPC)]
            qc = qb >> 4
            ob = pl.multiple_of(((q0 - (qc << 8)) << 7) + (qc << 5), 16)
            for u in range(16):
                acc0 = jnp.zeros((16,), jnp.float32)
                acc1 = jnp.zeros((16,), jnp.float32)
                for pc in range(_NPC):
                    base = pl.multiple_of(rows_v[pc][u], 16)
                    w = w_v[pc][u]
                    g0 = table[pl.ds(base, 16)]
                    g1 = table[pl.ds(base + 16, 16)]
                    acc0 = acc0 + w * g0
                    acc1 = acc1 + w * g1
                outv[pl.ds(ob + u * 128, 16)] = acc0
                outv[pl.ds(ob + u * 128 + 16, 16)] = acc1
            return qcarry

        lax.fori_loop(0, _NQ // 16, q_body, 0)
        pltpu.sync_copy(outv, out_hbm.at[pl.ds(r0, _NQ * _HD)])
        return carry

    lax.fori_loop(0, ppw, pair_body, 0)


def _sc_gather(vh, idx, wgt, nb):
    mesh = plsc.VectorSubcoreMesh(core_axis_name="c", subcore_axis_name="s",
                                  num_cores=_NC, num_subcores=_NS)
    ppw = (nb * _NH) // _NWORK
    return pl.kernel(
        functools.partial(_sc_body, ppw),
        out_type=jax.ShapeDtypeStruct((nb * _NH * _NQ * _HD,), jnp.float32),
        mesh=mesh,
        scratch_types=[
            pltpu.VMEM((_NQ * _HD,), jnp.float32),
            pltpu.VMEM((_NPC, _NQ), jnp.int32),
            pltpu.VMEM((_NPC, _NQ), jnp.float32),
            pltpu.VMEM((_NQ * _HD,), jnp.float32),
        ],
        compiler_params=pltpu.CompilerParams(needs_layout_passes=False),
    )(vh.reshape(-1), idx, wgt)


def _prep_call(query, value, W_val, b_val_r, W_off_r, b_off_r, W_attn_r,
               b_attn_r, b0, nb, *, interpret=False):
    full = lambda shape: pl.BlockSpec(shape, lambda b: (0,) * len(shape))
    return pl.pallas_call(
        _prep_body,
        grid=(nb,),
        in_specs=[
            pl.BlockSpec((1, _NQ, _D), lambda b: (b + b0, 0, 0)),
            pl.BlockSpec((1, _NQ, _D), lambda b: (b + b0, 0, 0)),
            full((_D, _D)),
            full((_NH, 1, _HD)),
            full((2 * _NP * _NH, _D)),
            full((2 * _NP * _NH, 1)),
            full((_NP * _NH, _D)),
            full((_NP * _NH, 1)),
        ],
        out_specs=[
            pl.BlockSpec((_NH * 256, 128), lambda b: (b, 0)),
            pl.BlockSpec((_NH, _NPC, _NQ), lambda b: (b, 0, 0)),
            pl.BlockSpec((_NH, _NPC, _NQ), lambda b: (b, 0, 0)),
        ],
        out_shape=[
            jax.ShapeDtypeStruct((nb * _NH * 256, 128), jnp.float32),
            jax.ShapeDtypeStruct((nb * _NH, _NPC, _NQ), jnp.int32),
            jax.ShapeDtypeStruct((nb * _NH, _NPC, _NQ), jnp.float32),
        ],
        interpret=interpret,
    )(query, value, W_val, b_val_r, W_off_r, b_off_r, W_attn_r, b_attn_r)


def _out_call(sh, W_out_r, b_out_r, nb, *, interpret=False):
    return pl.pallas_call(
        _out_body,
        grid=(nb,),
        in_specs=[
            pl.BlockSpec((_NH * 256, 128), lambda b: (b, 0)),
            pl.BlockSpec((_NH, _D, _HD), lambda b: (0, 0, 0)),
            pl.BlockSpec((1, _D), lambda b: (0, 0)),
        ],
        out_specs=pl.BlockSpec((1, _NQ, _D), lambda b: (b, 0, 0)),
        out_shape=jax.ShapeDtypeStruct((nb, _NQ, _D), jnp.float32),
        interpret=interpret,
    )(sh, W_out_r, b_out_r)


def kernel(query, value, W_off, b_off, W_attn, b_attn, W_val, b_val, W_out,
           b_out, spatial_shape, *, interpret=False):
    W_off_r = W_off.reshape(_NH, _NP, 2, _D).transpose(2, 1, 0, 3).reshape(2 * _NP * _NH, _D)
    b_off_r = b_off.reshape(_NH, _NP, 2).transpose(2, 1, 0).reshape(2 * _NP * _NH, 1)
    W_attn_r = W_attn.reshape(_NH, _NP, _D).transpose(1, 0, 2).reshape(_NP * _NH, _D)
    b_attn_r = b_attn.reshape(_NH, _NP).transpose(1, 0).reshape(_NP * _NH, 1)
    b_val_r = b_val.reshape(_NH, 1, _HD)
    W_out_r = W_out.reshape(_D, _NH, _HD).transpose(1, 0, 2)
    b_out_r = b_out.reshape(1, _D)

    ngroups = 4
    nb = _B // ngroups
    outs = []
    for g in range(ngroups):
        vh, idx, wgt = _prep_call(query, value, W_val, b_val_r, W_off_r,
                                  b_off_r, W_attn_r, b_attn_r, g * nb, nb,
                                  interpret=interpret)
        sh = _sc_gather(vh, idx, wgt, nb).reshape(nb * _NH * 256, 128)
        outs.append(_out_call(sh, W_out_r, b_out_r, nb, interpret=interpret))
    return jnp.concatenate(outs, axis=0)
